# baseline (speedup 1.0000x reference)
import functools

import jax
import jax.numpy as jnp
from jax import lax
from jax.experimental import pallas as pl
from jax.experimental.pallas import tpu as pltpu
from jax.experimental.pallas import tpu_sc as plsc

_N = 10000
_E = 320000
_NC = 2
_NS = 16
_NW = _NC * _NS
_CH = 128
_NCHUNK = 81
_EPW = _CH * _NCHUNK
_EPAD = _NW * _EPW
_NBUF = 3

_RPT = 624
_RPT_LAST = _N - (_NS - 1) * _RPT


def _mesh():
    return plsc.VectorSubcoreMesh(
        core_axis_name="c", subcore_axis_name="s", num_cores=_NC, num_subcores=_NS
    )


def _zero_vmem_2d(ref, nrows, d):
    nsl = d // 16

    def body(i, _):
        r = i // nsl
        cidx = (i % nsl) * 16
        ref[r, pl.ds(cidx, 16)] = jnp.zeros((16,), jnp.float32)
        return 0

    lax.fori_loop(0, nrows * nsl, body, 0)


def _zero_vmem_1d(ref, n):
    def body(i, _):
        ref[pl.ds(i * 16, 16)] = jnp.zeros((16,), jnp.float32)
        return 0

    lax.fori_loop(0, n // 16, body, 0)


@functools.cache
def _make_deg_kernel():
    return functools.partial(
        pl.kernel,
        out_type=jax.ShapeDtypeStruct((_NC * _N,), jnp.float32),
        mesh=_mesh(),
        scratch_types=[
            pltpu.VMEM((_NCHUNK, _CH), jnp.int32),
            pltpu.VMEM((_NCHUNK, _CH), jnp.float32),
            pltpu.VMEM((640,), jnp.float32),
            pltpu.VMEM_SHARED((_N,), jnp.float32),
            pltpu.SemaphoreType.DMA,
        ],
    )(_deg_body)


def _deg_body(dst_hbm, w_hbm, out_hbm, dst_v, w_v, zb, acc_sh, dsem):
    cid = lax.axis_index("c")
    sid = lax.axis_index("s")
    wid = cid * _NS + sid

    _zero_vmem_1d(zb, 640)
    off = sid * _RPT

    @pl.when(sid < _NS - 1)
    def _():
        pltpu.sync_copy(zb.at[pl.ds(0, _RPT)], acc_sh.at[pl.ds(off, _RPT)])

    @pl.when(sid == _NS - 1)
    def _():
        pltpu.sync_copy(zb, acc_sh.at[pl.ds(off, _RPT_LAST)])

    pltpu.sync_copy(dst_hbm.at[wid], dst_v)
    pltpu.sync_copy(w_hbm.at[wid], w_v)
    plsc.subcore_barrier()

    def group(gk, _):
        base = gk * 3
        for t in range(3):
            pltpu.async_copy(
                w_v.at[base + t], acc_sh.at[dst_v.at[base + t]], dsem, add=True
            )
        for t in range(3):
            pltpu.make_async_copy(w_v.at[base], acc_sh.at[dst_v.at[base]], dsem).wait()
        return 0

    lax.fori_loop(0, _NCHUNK // 3, group, 0)
    plsc.subcore_barrier()

    obase = cid * _N + off

    @pl.when(sid < _NS - 1)
    def _():
        pltpu.sync_copy(acc_sh.at[pl.ds(off, _RPT)], zb.at[pl.ds(0, _RPT)])
        pltpu.sync_copy(zb.at[pl.ds(0, _RPT)], out_hbm.at[pl.ds(obase, _RPT)])

    @pl.when(sid == _NS - 1)
    def _():
        pltpu.sync_copy(acc_sh.at[pl.ds(off, _RPT_LAST)], zb)
        pltpu.sync_copy(zb, out_hbm.at[pl.ds(obase, _RPT_LAST)])


@functools.cache
def _make_edge_kernel(d, ntab):
    nsl = d // 16

    @functools.partial(
        pl.kernel,
        out_type=[jax.ShapeDtypeStruct((_NC, _N, d), jnp.float32)] * ntab,
        mesh=_mesh(),
        scratch_types=[
            pltpu.VMEM((_NCHUNK, _CH), jnp.int32),
            pltpu.VMEM((_NCHUNK, _CH), jnp.int32),
            pltpu.VMEM((_NCHUNK, _CH), jnp.float32),
            *[pltpu.VMEM((_CH, d), jnp.float32) for _ in range(_NBUF)],
            *[pltpu.VMEM((_CH, d), jnp.float32) for _ in range(_NBUF)],
            pltpu.VMEM_SHARED((_N, d), jnp.float32),
            *[pltpu.SemaphoreType.DMA for _ in range(2 * _NBUF)],
        ],
        compiler_params=pltpu.CompilerParams(use_tc_tiling_on_sc=False),
    )
    def ek(*refs):
        t_hbms = refs[:ntab]
        src_hbm, dst_hbm, w_hbm = refs[ntab : ntab + 3]
        out_hbms = refs[ntab + 3 : 2 * ntab + 3]
        (src_v, dst_v, w_v) = refs[2 * ntab + 3 : 2 * ntab + 6]
        bufs = refs[2 * ntab + 6 :]
        rows = bufs[:_NBUF]
        srows = bufs[_NBUF : 2 * _NBUF]
        acc_sh = bufs[2 * _NBUF]
        gsem = bufs[2 * _NBUF + 1 : 2 * _NBUF + 1 + _NBUF]
        ssem = bufs[2 * _NBUF + 1 + _NBUF :]
        srows0_v = srows[0]
        rows0_v = rows[0]
        cid = lax.axis_index("c")
        sid = lax.axis_index("s")
        wid = cid * _NS + sid
        r0 = sid * _RPT

        pltpu.sync_copy(src_hbm.at[wid], src_v)
        pltpu.sync_copy(dst_hbm.at[wid], dst_v)
        pltpu.sync_copy(w_hbm.at[wid], w_v)

        def zero_acc():
            _zero_vmem_2d(srows0_v, _CH, d)

            @pl.when(sid < _NS - 1)
            def _():
                for k in range(_RPT // _CH):
                    pltpu.sync_copy(srows0_v, acc_sh.at[pl.ds(r0 + k * _CH, _CH)])
                rem = _RPT % _CH
                pltpu.sync_copy(
                    srows0_v.at[pl.ds(0, rem)],
                    acc_sh.at[pl.ds(r0 + (_RPT // _CH) * _CH, rem)],
                )

            @pl.when(sid == _NS - 1)
            def _():
                for k in range(_RPT_LAST // _CH):
                    pltpu.sync_copy(srows0_v, acc_sh.at[pl.ds(r0 + k * _CH, _CH)])

        def scale(j, par):
            rv = rows[par]
            sv = srows[par]

            def scale_group(g, _):
                wvec = w_v[j, pl.ds(g * 16, 16)]
                for lane in range(16):
                    e = g * 16 + lane
                    wl = wvec.at[jnp.full((16,), lane, jnp.int32)].get(
                        mode="promise_in_bounds"
                    )
                    for c in range(nsl):
                        sv[e, pl.ds(c * 16, 16)] = rv[e, pl.ds(c * 16, 16)] * wl
                return 0

            lax.fori_loop(0, _CH // 16, scale_group, 0)

        def run_pass(t_hbm, out_hbm):
            zero_acc()
            plsc.subcore_barrier()

            def start_gather(j, par):
                pltpu.async_copy(t_hbm.at[src_v.at[j]], rows[par], gsem[par])

            def wait_gather(j, par):
                pltpu.make_async_copy(
                    t_hbm.at[src_v.at[j]], rows[par], gsem[par]
                ).wait()

            def start_scatter(j, par):
                pltpu.async_copy(
                    srows[par], acc_sh.at[dst_v.at[j]], ssem[par], add=True
                )

            def wait_scatter(j, par):
                pltpu.make_async_copy(
                    srows[par], acc_sh.at[dst_v.at[j]], ssem[par]
                ).wait()

            for par in range(_NBUF):
                start_gather(par, par)

            def rotation(k, _):
                for par in range(_NBUF):
                    j = _NBUF * k + par
                    wait_gather(j, par)

                    @pl.when(k > 0)
                    def _():
                        wait_scatter(j, par)

                    scale(j, par)

                    @pl.when(j + _NBUF < _NCHUNK)
                    def _():
                        start_gather(j + _NBUF, par)

                    start_scatter(j, par)
                return 0

            lax.fori_loop(0, _NCHUNK // _NBUF, rotation, 0)

            for par in range(_NBUF):
                wait_scatter(0, par)
            plsc.subcore_barrier()

            @pl.when(sid < _NS - 1)
            def _():
                for k in range(_RPT // _CH):
                    pltpu.sync_copy(acc_sh.at[pl.ds(r0 + k * _CH, _CH)], rows0_v)
                    pltpu.sync_copy(
                        rows0_v, out_hbm.at[cid, pl.ds(r0 + k * _CH, _CH)]
                    )
                rem = _RPT % _CH
                ro = r0 + (_RPT // _CH) * _CH
                pltpu.sync_copy(acc_sh.at[pl.ds(ro, rem)], rows0_v.at[pl.ds(0, rem)])
                pltpu.sync_copy(
                    rows0_v.at[pl.ds(0, rem)], out_hbm.at[cid, pl.ds(ro, rem)]
                )

            @pl.when(sid == _NS - 1)
            def _():
                for k in range(_RPT_LAST // _CH):
                    pltpu.sync_copy(acc_sh.at[pl.ds(r0 + k * _CH, _CH)], rows0_v)
                    pltpu.sync_copy(
                        rows0_v, out_hbm.at[cid, pl.ds(r0 + k * _CH, _CH)]
                    )

        for tab in range(ntab):
            run_pass(t_hbms[tab], out_hbms[tab])

    return ek


def _edge_pass(t, src3, dst3, w3):
    d = t.shape[1]
    if d == 128:
        res = _make_edge_kernel(64, 2)(t[:, :64], t[:, 64:], src3, dst3, w3)
    else:
        res = _make_edge_kernel(64, 1)(t, src3, dst3, w3)
    return list(res) if isinstance(res, (list, tuple)) else [res]


def _tc_call(fn, out_shapes, *args):
    return pl.pallas_call(
        fn,
        out_shape=out_shapes,
        compiler_params=pltpu.CompilerParams(vmem_limit_bytes=100 * 1024 * 1024),
    )(*args)


def _prep_body(deg_ref, x_ref, w0_ref, b_ref, dis_ref, t0_ref, m0_ref):
    deg = deg_ref[0, :] + deg_ref[1, :]
    dis = jnp.where(deg > 0, lax.rsqrt(jnp.maximum(deg, 1e-12)), 0.0)
    dis2 = jnp.reshape(dis, (_N, 1))
    dis_ref[...] = dis2
    x = x_ref[...]
    t0_ref[...] = dis2 * x
    m0_ref[...] = jnp.dot(x, w0_ref[...], preferred_element_type=jnp.float32) + b_ref[0, :]


def _prep(deg2, x, w0, b):
    return _tc_call(
        _prep_body,
        (
            jax.ShapeDtypeStruct((_N, 1), jnp.float32),
            jax.ShapeDtypeStruct((_N, x.shape[1]), jnp.float32),
            jax.ShapeDtypeStruct((_N, w0.shape[1]), jnp.float32),
        ),
        deg2, x, w0, b.reshape(1, -1),
    )


def _bn_lrelu(h, gamma, beta):
    mean = jnp.mean(h, axis=0, keepdims=True)
    var = jnp.mean((h - mean) * (h - mean), axis=0, keepdims=True)
    h = (h - mean) * lax.rsqrt(var + 1e-5) * gamma + beta
    return jnp.where(h >= 0.0, h, 0.01 * h)


def _make_combine_body(post_w1, last, nz):
    def body(*refs):
        m_ref = refs[0]
        z_refs = refs[1 : 1 + nz]
        rest = refs[1 + nz :]
        if post_w1:
            (dis_ref, w1p_ref, g_ref, bt_ref,
             w1n_ref, w0n_ref, bn_ref, t_ref, mn_ref) = rest
        elif last:
            (dis_ref, g_ref, bt_ref, out_ref) = rest
        else:
            (dis_ref, g_ref, bt_ref,
             w1n_ref, w0n_ref, bn_ref, t_ref, mn_ref) = rest

        dis2 = dis_ref[...]
        z = jnp.concatenate([zr[0] + zr[1] for zr in z_refs], axis=-1)
        if post_w1:
            tx = (-dis2) * z
            add = jnp.dot(tx, w1p_ref[...], preferred_element_type=jnp.float32)
        else:
            add = dis2 * z
        h = m_ref[...] + add
        h = _bn_lrelu(h, g_ref[0, :], bt_ref[0, :])
        if last:
            out_ref[...] = h
        else:
            t_ref[...] = (-dis2) * jnp.dot(
                h, w1n_ref[...], preferred_element_type=jnp.float32
            )
            mn_ref[...] = (
                jnp.dot(h, w0n_ref[...], preferred_element_type=jnp.float32)
                + bn_ref[0, :]
            )

    return body


_combine_first = _make_combine_body(post_w1=True, last=False, nz=2)
_combine_mid2 = _make_combine_body(post_w1=False, last=False, nz=2)
_combine_mid1 = _make_combine_body(post_w1=False, last=False, nz=1)
_combine_last = _make_combine_body(post_w1=False, last=True, nz=1)


def kernel(x, edge_index, edge_attr, nroi, params):
    npad = _EPAD - _E
    pad_idx = jnp.arange(npad, dtype=jnp.int32) % _N
    src3 = jnp.concatenate([edge_index[0], pad_idx]).reshape(_NW, _NCHUNK, _CH)
    dst3 = jnp.concatenate([edge_index[1], pad_idx]).reshape(_NW, _NCHUNK, _CH)
    w3 = jnp.concatenate(
        [edge_attr, jnp.zeros((npad,), jnp.float32)]
    ).reshape(_NW, _NCHUNK, _CH)

    deg2 = _make_deg_kernel()(dst3, w3).reshape(_NC, _N)

    dis, t0, m0 = _prep(deg2, x, params["W0_0"], params["b_0"])

    z0 = _edge_pass(t0, src3, dst3, w3)
    t1, m1 = _tc_call(
        _combine_first,
        (
            jax.ShapeDtypeStruct((_N, 128), jnp.float32),
            jax.ShapeDtypeStruct((_N, 128), jnp.float32),
        ),
        m0, *z0, dis, params["W1_0"],
        params["gamma_0"].reshape(1, -1), params["beta_0"].reshape(1, -1),
        params["W1_1"], params["W0_1"], params["b_1"].reshape(1, -1),
    )

    z1 = _edge_pass(t1, src3, dst3, w3)
    t2, m2 = _tc_call(
        _combine_mid2,
        (
            jax.ShapeDtypeStruct((_N, 64), jnp.float32),
            jax.ShapeDtypeStruct((_N, 64), jnp.float32),
        ),
        m1, *z1, dis,
        params["gamma_1"].reshape(1, -1), params["beta_1"].reshape(1, -1),
        params["W1_2"], params["W0_2"], params["b_2"].reshape(1, -1),
    )

    z2 = _edge_pass(t2, src3, dst3, w3)
    t3, m3 = _tc_call(
        _combine_mid1,
        (
            jax.ShapeDtypeStruct((_N, 64), jnp.float32),
            jax.ShapeDtypeStruct((_N, 64), jnp.float32),
        ),
        m2, *z2, dis,
        params["gamma_2"].reshape(1, -1), params["beta_2"].reshape(1, -1),
        params["W1_3"], params["W0_3"], params["b_3"].reshape(1, -1),
    )

    z3 = _edge_pass(t3, src3, dst3, w3)
    out = _tc_call(
        _combine_last,
        jax.ShapeDtypeStruct((_N, 64), jnp.float32),
        m3, *z3, dis,
        params["gamma_3"].reshape(1, -1), params["beta_3"].reshape(1, -1),
    )
    return out


# speedup vs baseline: 1.0158x; 1.0158x over previous
import functools

import jax
import jax.numpy as jnp
from jax import lax
from jax.experimental import pallas as pl
from jax.experimental.pallas import tpu as pltpu
from jax.experimental.pallas import tpu_sc as plsc

_N = 10000
_E = 320000
_NC = 2
_NS = 16
_NW = _NC * _NS
_CH = 128
_NCHUNK = 81
_EPW = _CH * _NCHUNK
_EPAD = _NW * _EPW
_NBUF = 3

_RPT = 624
_RPT_LAST = _N - (_NS - 1) * _RPT


def _mesh():
    return plsc.VectorSubcoreMesh(
        core_axis_name="c", subcore_axis_name="s", num_cores=_NC, num_subcores=_NS
    )


def _zero_vmem_2d(ref, nrows, d):
    nsl = d // 16

    def body(i, _):
        r = i // nsl
        cidx = (i % nsl) * 16
        ref[r, pl.ds(cidx, 16)] = jnp.zeros((16,), jnp.float32)
        return 0

    lax.fori_loop(0, nrows * nsl, body, 0)


def _zero_vmem_1d(ref, n):
    def body(i, _):
        ref[pl.ds(i * 16, 16)] = jnp.zeros((16,), jnp.float32)
        return 0

    lax.fori_loop(0, n // 16, body, 0)


@functools.cache
def _make_deg_kernel():
    return functools.partial(
        pl.kernel,
        out_type=jax.ShapeDtypeStruct((_NC * _N,), jnp.float32),
        mesh=_mesh(),
        scratch_types=[
            pltpu.VMEM((_NCHUNK, _CH), jnp.int32),
            pltpu.VMEM((_NCHUNK, _CH), jnp.float32),
            pltpu.VMEM((640,), jnp.float32),
            pltpu.VMEM_SHARED((_N,), jnp.float32),
            pltpu.SemaphoreType.DMA,
        ],
    )(_deg_body)


def _deg_body(dst_hbm, w_hbm, out_hbm, dst_v, w_v, zb, acc_sh, dsem):
    cid = lax.axis_index("c")
    sid = lax.axis_index("s")
    wid = cid * _NS + sid

    _zero_vmem_1d(zb, 640)
    off = sid * _RPT

    @pl.when(sid < _NS - 1)
    def _():
        pltpu.sync_copy(zb.at[pl.ds(0, _RPT)], acc_sh.at[pl.ds(off, _RPT)])

    @pl.when(sid == _NS - 1)
    def _():
        pltpu.sync_copy(zb, acc_sh.at[pl.ds(off, _RPT_LAST)])

    pltpu.sync_copy(dst_hbm.at[wid], dst_v)
    pltpu.sync_copy(w_hbm.at[wid], w_v)
    plsc.subcore_barrier()

    def group(gk, _):
        base = gk * 3
        for t in range(3):
            pltpu.async_copy(
                w_v.at[base + t], acc_sh.at[dst_v.at[base + t]], dsem, add=True
            )
        for t in range(3):
            pltpu.make_async_copy(w_v.at[base], acc_sh.at[dst_v.at[base]], dsem).wait()
        return 0

    lax.fori_loop(0, _NCHUNK // 3, group, 0)
    plsc.subcore_barrier()

    obase = cid * _N + off

    @pl.when(sid < _NS - 1)
    def _():
        pltpu.sync_copy(acc_sh.at[pl.ds(off, _RPT)], zb.at[pl.ds(0, _RPT)])
        pltpu.sync_copy(zb.at[pl.ds(0, _RPT)], out_hbm.at[pl.ds(obase, _RPT)])

    @pl.when(sid == _NS - 1)
    def _():
        pltpu.sync_copy(acc_sh.at[pl.ds(off, _RPT_LAST)], zb)
        pltpu.sync_copy(zb, out_hbm.at[pl.ds(obase, _RPT_LAST)])


@functools.cache
def _make_edge_kernel(d, ntab):
    nsl = d // 16

    @functools.partial(
        pl.kernel,
        out_type=[jax.ShapeDtypeStruct((_NC, _N, d), jnp.float32)] * ntab,
        mesh=_mesh(),
        scratch_types=[
            pltpu.VMEM((_NCHUNK, _CH), jnp.int32),
            pltpu.VMEM((_NCHUNK, _CH), jnp.int32),
            pltpu.VMEM((_NCHUNK, _CH), jnp.float32),
            *[pltpu.VMEM((_CH, d), jnp.float32) for _ in range(_NBUF)],
            *[pltpu.VMEM((_CH, d), jnp.float32) for _ in range(_NBUF)],
            pltpu.VMEM_SHARED((_N, d), jnp.float32),
            *[pltpu.SemaphoreType.DMA for _ in range(2 * _NBUF)],
        ],
        compiler_params=pltpu.CompilerParams(use_tc_tiling_on_sc=False),
    )
    def ek(*refs):
        t_hbms = refs[:ntab]
        src_hbm, dst_hbm, w_hbm = refs[ntab : ntab + 3]
        out_hbms = refs[ntab + 3 : 2 * ntab + 3]
        (src_v, dst_v, w_v) = refs[2 * ntab + 3 : 2 * ntab + 6]
        bufs = refs[2 * ntab + 6 :]
        rows = bufs[:_NBUF]
        srows = bufs[_NBUF : 2 * _NBUF]
        acc_sh = bufs[2 * _NBUF]
        gsem = bufs[2 * _NBUF + 1 : 2 * _NBUF + 1 + _NBUF]
        ssem = bufs[2 * _NBUF + 1 + _NBUF :]
        srows0_v = srows[0]
        rows0_v = rows[0]
        cid = lax.axis_index("c")
        sid = lax.axis_index("s")
        wid = cid * _NS + sid
        r0 = sid * _RPT

        pltpu.async_copy(src_hbm.at[wid], src_v, gsem[0])
        pltpu.async_copy(dst_hbm.at[wid], dst_v, gsem[1])
        pltpu.async_copy(w_hbm.at[wid], w_v, gsem[2])
        pltpu.make_async_copy(src_hbm.at[wid], src_v, gsem[0]).wait()
        pltpu.make_async_copy(dst_hbm.at[wid], dst_v, gsem[1]).wait()
        pltpu.make_async_copy(w_hbm.at[wid], w_v, gsem[2]).wait()

        def zero_acc():
            _zero_vmem_2d(srows0_v, _CH, d)

            @pl.when(sid < _NS - 1)
            def _():
                for k in range(_RPT // _CH):
                    pltpu.async_copy(
                        srows0_v, acc_sh.at[pl.ds(r0 + k * _CH, _CH)], gsem[0]
                    )
                rem = _RPT % _CH
                pltpu.async_copy(
                    srows0_v.at[pl.ds(0, rem)],
                    acc_sh.at[pl.ds(r0 + (_RPT // _CH) * _CH, rem)],
                    gsem[0],
                )
                for k in range(_RPT // _CH):
                    pltpu.make_async_copy(
                        srows0_v, acc_sh.at[pl.ds(r0 + k * _CH, _CH)], gsem[0]
                    ).wait()
                pltpu.make_async_copy(
                    srows0_v.at[pl.ds(0, rem)],
                    acc_sh.at[pl.ds(r0 + (_RPT // _CH) * _CH, rem)],
                    gsem[0],
                ).wait()

            @pl.when(sid == _NS - 1)
            def _():
                for k in range(_RPT_LAST // _CH):
                    pltpu.async_copy(
                        srows0_v, acc_sh.at[pl.ds(r0 + k * _CH, _CH)], gsem[0]
                    )
                for k in range(_RPT_LAST // _CH):
                    pltpu.make_async_copy(
                        srows0_v, acc_sh.at[pl.ds(r0 + k * _CH, _CH)], gsem[0]
                    ).wait()

        def scale(j, par):
            rv = rows[par]
            sv = srows[par]

            def scale_group(g, _):
                wvec = w_v[j, pl.ds(g * 16, 16)]
                for lane in range(16):
                    e = g * 16 + lane
                    wl = wvec.at[jnp.full((16,), lane, jnp.int32)].get(
                        mode="promise_in_bounds"
                    )
                    for c in range(nsl):
                        sv[e, pl.ds(c * 16, 16)] = rv[e, pl.ds(c * 16, 16)] * wl
                return 0

            lax.fori_loop(0, _CH // 16, scale_group, 0)

        def run_pass(t_hbm, out_hbm):
            zero_acc()
            plsc.subcore_barrier()

            def start_gather(j, par):
                pltpu.async_copy(t_hbm.at[src_v.at[j]], rows[par], gsem[par])

            def wait_gather(j, par):
                pltpu.make_async_copy(
                    t_hbm.at[src_v.at[j]], rows[par], gsem[par]
                ).wait()

            def start_scatter(j, par):
                pltpu.async_copy(
                    srows[par], acc_sh.at[dst_v.at[j]], ssem[par], add=True
                )

            def wait_scatter(j, par):
                pltpu.make_async_copy(
                    srows[par], acc_sh.at[dst_v.at[j]], ssem[par]
                ).wait()

            for par in range(_NBUF):
                start_gather(par, par)

            def rotation(k, _):
                for par in range(_NBUF):
                    j = _NBUF * k + par
                    wait_gather(j, par)

                    @pl.when(k > 0)
                    def _():
                        wait_scatter(j, par)

                    scale(j, par)

                    @pl.when(j + _NBUF < _NCHUNK)
                    def _():
                        start_gather(j + _NBUF, par)

                    start_scatter(j, par)
                return 0

            lax.fori_loop(0, _NCHUNK // _NBUF, rotation, 0)

            for par in range(_NBUF):
                wait_scatter(0, par)
            plsc.subcore_barrier()

            bb = list(rows) + list(srows)

            @pl.when(sid < _NS - 1)
            def _():
                nk = _RPT // _CH
                rem = _RPT % _CH
                ro = r0 + nk * _CH
                for k in range(nk):
                    pltpu.async_copy(
                        acc_sh.at[pl.ds(r0 + k * _CH, _CH)], bb[k], gsem[0]
                    )
                pltpu.async_copy(
                    acc_sh.at[pl.ds(ro, rem)], bb[nk].at[pl.ds(0, rem)], gsem[0]
                )
                for k in range(nk):
                    pltpu.make_async_copy(
                        acc_sh.at[pl.ds(r0 + k * _CH, _CH)], bb[k], gsem[0]
                    ).wait()
                pltpu.make_async_copy(
                    acc_sh.at[pl.ds(ro, rem)], bb[nk].at[pl.ds(0, rem)], gsem[0]
                ).wait()
                for k in range(nk):
                    pltpu.async_copy(
                        bb[k], out_hbm.at[cid, pl.ds(r0 + k * _CH, _CH)], ssem[0]
                    )
                pltpu.async_copy(
                    bb[nk].at[pl.ds(0, rem)], out_hbm.at[cid, pl.ds(ro, rem)], ssem[0]
                )
                for k in range(nk):
                    pltpu.make_async_copy(
                        bb[k], out_hbm.at[cid, pl.ds(r0 + k * _CH, _CH)], ssem[0]
                    ).wait()
                pltpu.make_async_copy(
                    bb[nk].at[pl.ds(0, rem)], out_hbm.at[cid, pl.ds(ro, rem)], ssem[0]
                ).wait()

            @pl.when(sid == _NS - 1)
            def _():
                nk = _RPT_LAST // _CH
                for k in range(nk):
                    pltpu.async_copy(
                        acc_sh.at[pl.ds(r0 + k * _CH, _CH)], bb[k], gsem[0]
                    )
                for k in range(nk):
                    pltpu.make_async_copy(
                        acc_sh.at[pl.ds(r0 + k * _CH, _CH)], bb[k], gsem[0]
                    ).wait()
                for k in range(nk):
                    pltpu.async_copy(
                        bb[k], out_hbm.at[cid, pl.ds(r0 + k * _CH, _CH)], ssem[0]
                    )
                for k in range(nk):
                    pltpu.make_async_copy(
                        bb[k], out_hbm.at[cid, pl.ds(r0 + k * _CH, _CH)], ssem[0]
                    ).wait()

        for tab in range(ntab):
            run_pass(t_hbms[tab], out_hbms[tab])

    return ek


def _edge_pass(t, src3, dst3, w3):
    d = t.shape[1]
    if d == 128:
        res = _make_edge_kernel(64, 2)(t[:, :64], t[:, 64:], src3, dst3, w3)
    else:
        res = _make_edge_kernel(64, 1)(t, src3, dst3, w3)
    return list(res) if isinstance(res, (list, tuple)) else [res]


def _tc_call(fn, out_shapes, *args):
    return pl.pallas_call(
        fn,
        out_shape=out_shapes,
        compiler_params=pltpu.CompilerParams(vmem_limit_bytes=100 * 1024 * 1024),
    )(*args)


def _prep_body(deg_ref, x_ref, w0_ref, b_ref, dis_ref, t0_ref, m0_ref):
    deg = deg_ref[0, :] + deg_ref[1, :]
    dis = jnp.where(deg > 0, lax.rsqrt(jnp.maximum(deg, 1e-12)), 0.0)
    dis2 = jnp.reshape(dis, (_N, 1))
    dis_ref[...] = dis2
    x = x_ref[...]
    t0_ref[...] = dis2 * x
    m0_ref[...] = jnp.dot(x, w0_ref[...], preferred_element_type=jnp.float32) + b_ref[0, :]


def _prep(deg2, x, w0, b):
    return _tc_call(
        _prep_body,
        (
            jax.ShapeDtypeStruct((_N, 1), jnp.float32),
            jax.ShapeDtypeStruct((_N, x.shape[1]), jnp.float32),
            jax.ShapeDtypeStruct((_N, w0.shape[1]), jnp.float32),
        ),
        deg2, x, w0, b.reshape(1, -1),
    )


def _bn_lrelu(h, gamma, beta):
    mean = jnp.mean(h, axis=0, keepdims=True)
    var = jnp.mean((h - mean) * (h - mean), axis=0, keepdims=True)
    h = (h - mean) * lax.rsqrt(var + 1e-5) * gamma + beta
    return jnp.where(h >= 0.0, h, 0.01 * h)


def _make_combine_body(post_w1, last, nz):
    def body(*refs):
        m_ref = refs[0]
        z_refs = refs[1 : 1 + nz]
        rest = refs[1 + nz :]
        if post_w1:
            (dis_ref, w1p_ref, g_ref, bt_ref,
             w1n_ref, w0n_ref, bn_ref, t_ref, mn_ref) = rest
        elif last:
            (dis_ref, g_ref, bt_ref, out_ref) = rest
        else:
            (dis_ref, g_ref, bt_ref,
             w1n_ref, w0n_ref, bn_ref, t_ref, mn_ref) = rest

        dis2 = dis_ref[...]
        z = jnp.concatenate([zr[0] + zr[1] for zr in z_refs], axis=-1)
        if post_w1:
            tx = (-dis2) * z
            add = jnp.dot(tx, w1p_ref[...], preferred_element_type=jnp.float32)
        else:
            add = dis2 * z
        h = m_ref[...] + add
        h = _bn_lrelu(h, g_ref[0, :], bt_ref[0, :])
        if last:
            out_ref[...] = h
        else:
            t_ref[...] = (-dis2) * jnp.dot(
                h, w1n_ref[...], preferred_element_type=jnp.float32
            )
            mn_ref[...] = (
                jnp.dot(h, w0n_ref[...], preferred_element_type=jnp.float32)
                + bn_ref[0, :]
            )

    return body


_combine_first = _make_combine_body(post_w1=True, last=False, nz=2)
_combine_mid2 = _make_combine_body(post_w1=False, last=False, nz=2)
_combine_mid1 = _make_combine_body(post_w1=False, last=False, nz=1)
_combine_last = _make_combine_body(post_w1=False, last=True, nz=1)


def kernel(x, edge_index, edge_attr, nroi, params):
    npad = _EPAD - _E
    pad_idx = jnp.arange(npad, dtype=jnp.int32) % _N
    src3 = jnp.concatenate([edge_index[0], pad_idx]).reshape(_NW, _NCHUNK, _CH)
    dst3 = jnp.concatenate([edge_index[1], pad_idx]).reshape(_NW, _NCHUNK, _CH)
    w3 = jnp.concatenate(
        [edge_attr, jnp.zeros((npad,), jnp.float32)]
    ).reshape(_NW, _NCHUNK, _CH)

    deg2 = _make_deg_kernel()(dst3, w3).reshape(_NC, _N)

    dis, t0, m0 = _prep(deg2, x, params["W0_0"], params["b_0"])

    z0 = _edge_pass(t0, src3, dst3, w3)
    t1, m1 = _tc_call(
        _combine_first,
        (
            jax.ShapeDtypeStruct((_N, 128), jnp.float32),
            jax.ShapeDtypeStruct((_N, 128), jnp.float32),
        ),
        m0, *z0, dis, params["W1_0"],
        params["gamma_0"].reshape(1, -1), params["beta_0"].reshape(1, -1),
        params["W1_1"], params["W0_1"], params["b_1"].reshape(1, -1),
    )

    z1 = _edge_pass(t1, src3, dst3, w3)
    t2, m2 = _tc_call(
        _combine_mid2,
        (
            jax.ShapeDtypeStruct((_N, 64), jnp.float32),
            jax.ShapeDtypeStruct((_N, 64), jnp.float32),
        ),
        m1, *z1, dis,
        params["gamma_1"].reshape(1, -1), params["beta_1"].reshape(1, -1),
        params["W1_2"], params["W0_2"], params["b_2"].reshape(1, -1),
    )

    z2 = _edge_pass(t2, src3, dst3, w3)
    t3, m3 = _tc_call(
        _combine_mid1,
        (
            jax.ShapeDtypeStruct((_N, 64), jnp.float32),
            jax.ShapeDtypeStruct((_N, 64), jnp.float32),
        ),
        m2, *z2, dis,
        params["gamma_2"].reshape(1, -1), params["beta_2"].reshape(1, -1),
        params["W1_3"], params["W0_3"], params["b_3"].reshape(1, -1),
    )

    z3 = _edge_pass(t3, src3, dst3, w3)
    out = _tc_call(
        _combine_last,
        jax.ShapeDtypeStruct((_N, 64), jnp.float32),
        m3, *z3, dis,
        params["gamma_3"].reshape(1, -1), params["beta_3"].reshape(1, -1),
    )
    return out
